# baseline (speedup 1.0000x reference)
import jax
import jax.numpy as jnp
from jax import lax
from jax.experimental import pallas as pl
from jax.experimental.pallas import tpu as pltpu

EMB = 50
BATCH = 16384
GRID = 32
RPB = BATCH // GRID
NSEM = 8
VG = 125000
BG = 7813


def _tc_body(gidx_s, pidx_s, gidx_v, pidx_v, emb1, emb2, b1, b2, out_ref,
             g_grp, p_grp, bw1, bw2, *sems):
    def fire(g, carry):
        for t in range(NSEM):
            r = g * NSEM + t
            ridx = gidx_s[0, 0, r]
            pltpu.make_async_copy(
                emb1.at[ridx >> 3], g_grp.at[r], sems[t]).start()
            pltpu.make_async_copy(
                b1.at[ridx >> 7], bw1.at[r], sems[t]).start()
            ridx2 = pidx_s[0, 0, r]
            pltpu.make_async_copy(
                emb2.at[ridx2 >> 3], p_grp.at[r], sems[t]).start()
            pltpu.make_async_copy(
                b2.at[ridx2 >> 7], bw2.at[r], sems[t]).start()
        return carry

    lax.fori_loop(0, RPB // NSEM, fire, 0)

    per_sem = RPB // NSEM
    for t in range(NSEM):
        pltpu.make_async_copy(
            emb1.at[pl.ds(0, per_sem)], g_grp.at[pl.ds(0, per_sem)],
            sems[t]).wait()
        pltpu.make_async_copy(
            emb2.at[pl.ds(0, per_sem)], p_grp.at[pl.ds(0, per_sem)],
            sems[t]).wait()
        pltpu.make_async_copy(
            b1.at[pl.ds(0, per_sem)], bw1.at[pl.ds(0, per_sem)],
            sems[t]).wait()
        pltpu.make_async_copy(
            b2.at[pl.ds(0, per_sem)], bw2.at[pl.ds(0, per_sem)],
            sems[t]).wait()

    gi = gidx_v[0, 0, :]
    pi = pidx_v[0, 0, :]
    sub8 = lax.broadcasted_iota(jnp.int32, (RPB, 8, 1), 1)
    g_sel = jnp.sum(
        jnp.where((gi[:, None, None] & 7) == sub8, g_grp[...], 0.0), axis=1)
    p_sel = jnp.sum(
        jnp.where((pi[:, None, None] & 7) == sub8, p_grp[...], 0.0), axis=1)
    dots = jnp.sum(g_sel * p_sel, axis=1)

    lane = lax.broadcasted_iota(jnp.int32, (RPB, 128), 1)
    bias1 = jnp.sum(
        jnp.where((gi[:, None] & 127) == lane, bw1[:, 0, :], 0.0), axis=1)
    bias2 = jnp.sum(
        jnp.where((pi[:, None] & 127) == lane, bw2[:, 0, :], 0.0), axis=1)

    out_ref[0, 0, :] = dots + bias1 + bias2


@jax.jit
def _tc_call(gidx, pidx, emb_1, emb_2, b1, b2):
    emb1g = emb_1.reshape(VG, 8, EMB)
    emb2g = emb_2.reshape(VG, 8, EMB)
    b1w = jnp.pad(b1, (0, BG * 128 - b1.shape[0])).reshape(BG, 1, 128)
    b2w = jnp.pad(b2, (0, BG * 128 - b2.shape[0])).reshape(BG, 1, 128)
    gidx3 = gidx.reshape(GRID, 1, RPB)
    pidx3 = pidx.reshape(GRID, 1, RPB)
    grid_spec = pltpu.PrefetchScalarGridSpec(
        num_scalar_prefetch=0,
        grid=(GRID,),
        in_specs=[
            pl.BlockSpec((1, 1, RPB), lambda i: (i, 0, 0),
                         memory_space=pltpu.SMEM),
            pl.BlockSpec((1, 1, RPB), lambda i: (i, 0, 0),
                         memory_space=pltpu.SMEM),
            pl.BlockSpec((1, 1, RPB), lambda i: (i, 0, 0)),
            pl.BlockSpec((1, 1, RPB), lambda i: (i, 0, 0)),
            pl.BlockSpec(memory_space=pl.ANY),
            pl.BlockSpec(memory_space=pl.ANY),
            pl.BlockSpec(memory_space=pl.ANY),
            pl.BlockSpec(memory_space=pl.ANY),
        ],
        out_specs=pl.BlockSpec((1, 1, RPB), lambda i: (i, 0, 0)),
        scratch_shapes=[
            pltpu.VMEM((RPB, 8, EMB), jnp.float32),
            pltpu.VMEM((RPB, 8, EMB), jnp.float32),
            pltpu.VMEM((RPB, 1, 128), jnp.float32),
            pltpu.VMEM((RPB, 1, 128), jnp.float32),
        ] + [pltpu.SemaphoreType.DMA] * NSEM,
    )
    fn = pl.pallas_call(
        _tc_body,
        grid_spec=grid_spec,
        out_shape=jax.ShapeDtypeStruct((GRID, 1, RPB), jnp.float32),
    )
    out = fn(gidx3, pidx3, gidx3, pidx3, emb1g, emb2g, b1w, b2w)
    return out.reshape(BATCH)


def kernel(x, emb_1, emb_2, emb_1_bias, emb_2_bias):
    gidx = x[:, 0].astype(jnp.int32)
    pidx = x[:, 1].astype(jnp.int32)
    b1 = emb_1_bias.reshape(-1)
    b2 = emb_2_bias.reshape(-1)
    return _tc_call(gidx, pidx, emb_1, emb_2, b1, b2)


# speedup vs baseline: 1.8138x; 1.8138x over previous
import jax
import jax.numpy as jnp
from jax import lax
from jax.experimental import pallas as pl
from jax.experimental.pallas import tpu as pltpu

EMB = 50
BATCH = 16384
GRID = 32
RPB = BATCH // GRID
NSEM = 8


def _tc_body(gidx_s, pidx_s, gidx_v, pidx_v, emb1, emb2, b1, b2, out_ref,
             g_grp, p_grp, bw1, bw2, *sems):
    def fire(g, carry):
        for t in range(NSEM):
            r = g * NSEM + t
            ridx = gidx_s[0, 0, r]
            g8 = pl.multiple_of((ridx >> 3) << 3, 8)
            gw = pl.multiple_of((ridx >> 7) << 7, 128)
            pltpu.make_async_copy(
                emb1.at[pl.ds(g8, 8), :], g_grp.at[r], sems[t]).start()
            pltpu.make_async_copy(
                b1.at[pl.ds(gw, 128)], bw1.at[r], sems[t]).start()
            ridx2 = pidx_s[0, 0, r]
            p8 = pl.multiple_of((ridx2 >> 3) << 3, 8)
            pw = pl.multiple_of((ridx2 >> 7) << 7, 128)
            pltpu.make_async_copy(
                emb2.at[pl.ds(p8, 8), :], p_grp.at[r], sems[t]).start()
            pltpu.make_async_copy(
                b2.at[pl.ds(pw, 128)], bw2.at[r], sems[t]).start()
        return carry

    lax.fori_loop(0, RPB // NSEM, fire, 0)

    per_sem = RPB // NSEM
    for t in range(NSEM):
        gsl = g_grp.at[pl.ds(0, per_sem)]
        psl = p_grp.at[pl.ds(0, per_sem)]
        bsl1 = bw1.at[pl.ds(0, per_sem)]
        bsl2 = bw2.at[pl.ds(0, per_sem)]
        pltpu.make_async_copy(gsl, gsl, sems[t]).wait()
        pltpu.make_async_copy(psl, psl, sems[t]).wait()
        pltpu.make_async_copy(bsl1, bsl1, sems[t]).wait()
        pltpu.make_async_copy(bsl2, bsl2, sems[t]).wait()

    gi = gidx_v[0, 0, :]
    pi = pidx_v[0, 0, :]
    sub8 = lax.broadcasted_iota(jnp.int32, (RPB, 8, 1), 1)
    g_sel = jnp.sum(
        jnp.where((gi[:, None, None] & 7) == sub8, g_grp[...], 0.0), axis=1)
    p_sel = jnp.sum(
        jnp.where((pi[:, None, None] & 7) == sub8, p_grp[...], 0.0), axis=1)
    dots = jnp.sum(g_sel * p_sel, axis=1)

    lane = lax.broadcasted_iota(jnp.int32, (RPB, 128), 1)
    bias1 = jnp.sum(
        jnp.where((gi[:, None] & 127) == lane, bw1[...], 0.0), axis=1)
    bias2 = jnp.sum(
        jnp.where((pi[:, None] & 127) == lane, bw2[...], 0.0), axis=1)

    out_ref[0, 0, :] = dots + bias1 + bias2


@jax.jit
def _tc_call(gidx, pidx, emb_1, emb_2, b1, b2):
    gidx3 = gidx.reshape(GRID, 1, RPB)
    pidx3 = pidx.reshape(GRID, 1, RPB)
    grid_spec = pltpu.PrefetchScalarGridSpec(
        num_scalar_prefetch=0,
        grid=(GRID,),
        in_specs=[
            pl.BlockSpec((1, 1, RPB), lambda i: (i, 0, 0),
                         memory_space=pltpu.SMEM),
            pl.BlockSpec((1, 1, RPB), lambda i: (i, 0, 0),
                         memory_space=pltpu.SMEM),
            pl.BlockSpec((1, 1, RPB), lambda i: (i, 0, 0)),
            pl.BlockSpec((1, 1, RPB), lambda i: (i, 0, 0)),
            pl.BlockSpec(memory_space=pl.ANY),
            pl.BlockSpec(memory_space=pl.ANY),
            pl.BlockSpec(memory_space=pl.ANY),
            pl.BlockSpec(memory_space=pl.ANY),
        ],
        out_specs=pl.BlockSpec((1, 1, RPB), lambda i: (i, 0, 0)),
        scratch_shapes=[
            pltpu.VMEM((RPB, 8, EMB), jnp.float32),
            pltpu.VMEM((RPB, 8, EMB), jnp.float32),
            pltpu.VMEM((RPB, 128), jnp.float32),
            pltpu.VMEM((RPB, 128), jnp.float32),
        ] + [pltpu.SemaphoreType.DMA] * NSEM,
    )
    fn = pl.pallas_call(
        _tc_body,
        grid_spec=grid_spec,
        out_shape=jax.ShapeDtypeStruct((GRID, 1, RPB), jnp.float32),
    )
    out = fn(gidx3, pidx3, gidx3, pidx3, emb_1, emb_2, b1, b2)
    return out.reshape(BATCH)


def kernel(x, emb_1, emb_2, emb_1_bias, emb_2_bias):
    gidx = x[:, 0].astype(jnp.int32)
    pidx = x[:, 1].astype(jnp.int32)
    b1 = emb_1_bias.reshape(-1)
    b2 = emb_2_bias.reshape(-1)
    return _tc_call(gidx, pidx, emb_1, emb_2, b1, b2)


# speedup vs baseline: 2.2840x; 1.2593x over previous
import functools

import jax
import jax.numpy as jnp
from jax import lax
from jax.experimental import pallas as pl
from jax.experimental.pallas import tpu as pltpu
from jax.experimental.pallas import tpu_sc as plsc

EMB = 50
BATCH = 16384
SC_ROWS = 8192
NC, NS, L = 2, 16, 16
NW = NC * NS
BPW = BATCH // NW
TBW = SC_ROWS // NW
IG = 128
NG = BPW // IG
TNG = TBW // IG

GRID = 16
RPB = (BATCH - SC_ROWS) // GRID
NSEM = 8


def _sc_body(gidx_hbm, pidx_hbm, emb1_hbm, emb2_hbm, b1_hbm, b2_hbm,
             bias_out, dots_out, gidx_v, pidx_v, tg_v, tp_v, g_v, p_v,
             b1_v, b2_v, outb_v, outd_v, sem, rsem):
    wid = lax.axis_index("s") * NC + lax.axis_index("c")
    bbase = wid * BPW
    tbase = wid * TBW

    for k in range(NG):
        pltpu.sync_copy(gidx_hbm.at[pl.ds(bbase + k * IG, IG)], gidx_v.at[k])
        pltpu.sync_copy(pidx_hbm.at[pl.ds(bbase + k * IG, IG)], pidx_v.at[k])
    for k in range(TNG):
        pltpu.sync_copy(gidx_hbm.at[pl.ds(tbase + k * IG, IG)], tg_v.at[k])
        pltpu.sync_copy(pidx_hbm.at[pl.ds(tbase + k * IG, IG)], tp_v.at[k])

    bias_copies = []
    for k in range(NG):
        sl = pl.ds(k * IG, IG)
        bias_copies.append(pltpu.async_copy(
            b1_hbm.at[gidx_v.at[k]], b1_v.at[sl], sem))
        bias_copies.append(pltpu.async_copy(
            b2_hbm.at[pidx_v.at[k]], b2_v.at[sl], sem))

    lanes = lax.iota(jnp.int32, L)

    def fire(grp, c):
        k = grp // (IG // L)
        i = (grp % (IG // L)) * L
        gv16 = tg_v[k, pl.ds(i, L)]
        pv16 = tp_v[k, pl.ds(i, L)]
        r0 = grp * L
        for t in range(L):
            pltpu.async_copy(emb1_hbm.at[gv16[t]], g_v.at[r0 + t], rsem)
            pltpu.async_copy(emb2_hbm.at[pv16[t]], p_v.at[r0 + t], rsem)
        return c

    lax.fori_loop(0, TBW // L, fire, 0)

    for cp in bias_copies:
        cp.wait()

    def bchunk(c, carry):
        r0 = c * L
        outb_v[pl.ds(r0, L)] = b1_v[pl.ds(r0, L)] + b2_v[pl.ds(r0, L)]
        return carry

    lax.fori_loop(0, BPW // L, bchunk, 0)
    pltpu.sync_copy(outb_v, bias_out.at[pl.ds(bbase, BPW)])

    pltpu.make_async_copy(
        emb1_hbm.at[pl.ds(0, TBW)], g_v, rsem).wait()
    pltpu.make_async_copy(
        emb2_hbm.at[pl.ds(0, TBW)], p_v, rsem).wait()

    def chunk(c, carry):
        r0 = c * L
        rows = r0 + lanes
        acc = jnp.zeros((L,), jnp.float32)
        for j in range(EMB):
            jv = jnp.full((L,), j, jnp.int32)
            gv = plsc.load_gather(g_v, [rows, jv])
            pv = plsc.load_gather(p_v, [rows, jv])
            acc = acc + gv * pv
        outd_v[pl.ds(r0, L)] = acc
        return carry

    lax.fori_loop(0, TBW // L, chunk, 0)
    pltpu.sync_copy(outd_v, dots_out.at[pl.ds(tbase, TBW)])


def _sc_call(gidx, pidx, emb_1, emb_2, b1, b2):
    mesh = plsc.VectorSubcoreMesh(core_axis_name="c", subcore_axis_name="s")
    k = functools.partial(
        pl.kernel,
        mesh=mesh,
        out_type=(
            jax.ShapeDtypeStruct((BATCH,), jnp.float32),
            jax.ShapeDtypeStruct((SC_ROWS,), jnp.float32),
        ),
        scratch_types=[
            pltpu.VMEM((NG, IG), jnp.int32),
            pltpu.VMEM((NG, IG), jnp.int32),
            pltpu.VMEM((TNG, IG), jnp.int32),
            pltpu.VMEM((TNG, IG), jnp.int32),
            pltpu.VMEM((TBW, EMB), jnp.float32),
            pltpu.VMEM((TBW, EMB), jnp.float32),
            pltpu.VMEM((BPW,), jnp.float32),
            pltpu.VMEM((BPW,), jnp.float32),
            pltpu.VMEM((BPW,), jnp.float32),
            pltpu.VMEM((TBW,), jnp.float32),
            pltpu.SemaphoreType.DMA,
            pltpu.SemaphoreType.DMA,
        ],
        compiler_params=pltpu.CompilerParams(needs_layout_passes=False),
    )(_sc_body)
    return k(gidx, pidx, emb_1, emb_2, b1, b2)


def _tc_body(gidx_s, pidx_s, gidx_v, pidx_v, emb1, emb2, out_ref,
             g_grp, p_grp, *sems):
    def fire(g, carry):
        for t in range(NSEM):
            r = g * NSEM + t
            ridx = gidx_s[0, 0, r]
            g8 = pl.multiple_of((ridx >> 3) << 3, 8)
            pltpu.make_async_copy(
                emb1.at[pl.ds(g8, 8), :], g_grp.at[r], sems[t]).start()
            ridx2 = pidx_s[0, 0, r]
            p8 = pl.multiple_of((ridx2 >> 3) << 3, 8)
            pltpu.make_async_copy(
                emb2.at[pl.ds(p8, 8), :], p_grp.at[r], sems[t]).start()
        return carry

    lax.fori_loop(0, RPB // NSEM, fire, 0)

    per_sem = RPB // NSEM
    for t in range(NSEM):
        gsl = g_grp.at[pl.ds(0, per_sem)]
        psl = p_grp.at[pl.ds(0, per_sem)]
        pltpu.make_async_copy(gsl, gsl, sems[t]).wait()
        pltpu.make_async_copy(psl, psl, sems[t]).wait()

    gi = gidx_v[0, 0, :]
    pi = pidx_v[0, 0, :]
    sub8 = lax.broadcasted_iota(jnp.int32, (RPB, 8, 1), 1)
    g_sel = jnp.sum(
        jnp.where((gi[:, None, None] & 7) == sub8, g_grp[...], 0.0), axis=1)
    p_sel = jnp.sum(
        jnp.where((pi[:, None, None] & 7) == sub8, p_grp[...], 0.0), axis=1)
    out_ref[0, 0, :] = jnp.sum(g_sel * p_sel, axis=1)


def _tc_call(gidx, pidx, emb_1, emb_2):
    gidx3 = gidx.reshape(GRID, 1, RPB)
    pidx3 = pidx.reshape(GRID, 1, RPB)
    grid_spec = pltpu.PrefetchScalarGridSpec(
        num_scalar_prefetch=0,
        grid=(GRID,),
        in_specs=[
            pl.BlockSpec((1, 1, RPB), lambda i: (i, 0, 0),
                         memory_space=pltpu.SMEM),
            pl.BlockSpec((1, 1, RPB), lambda i: (i, 0, 0),
                         memory_space=pltpu.SMEM),
            pl.BlockSpec((1, 1, RPB), lambda i: (i, 0, 0)),
            pl.BlockSpec((1, 1, RPB), lambda i: (i, 0, 0)),
            pl.BlockSpec(memory_space=pl.ANY),
            pl.BlockSpec(memory_space=pl.ANY),
        ],
        out_specs=pl.BlockSpec((1, 1, RPB), lambda i: (i, 0, 0)),
        scratch_shapes=[
            pltpu.VMEM((RPB, 8, EMB), jnp.float32),
            pltpu.VMEM((RPB, 8, EMB), jnp.float32),
        ] + [pltpu.SemaphoreType.DMA] * NSEM,
    )
    fn = pl.pallas_call(
        _tc_body,
        grid_spec=grid_spec,
        out_shape=jax.ShapeDtypeStruct((GRID, 1, RPB), jnp.float32),
    )
    out = fn(gidx3, pidx3, gidx3, pidx3, emb_1, emb_2)
    return out.reshape(BATCH - SC_ROWS)


def kernel(x, emb_1, emb_2, emb_1_bias, emb_2_bias):
    gidx = x[:, 0].astype(jnp.int32)
    pidx = x[:, 1].astype(jnp.int32)
    b1 = emb_1_bias.reshape(-1)
    b2 = emb_2_bias.reshape(-1)
    bias_sum, dots_sc = _sc_call(gidx, pidx, emb_1, emb_2, b1, b2)
    dots_tc = _tc_call(gidx[SC_ROWS:], pidx[SC_ROWS:], emb_1, emb_2)
    return jnp.concatenate(
        [dots_sc + bias_sum[:SC_ROWS], dots_tc + bias_sum[SC_ROWS:]])


# speedup vs baseline: 2.4935x; 1.0917x over previous
import functools

import jax
import jax.numpy as jnp
from jax import lax
from jax.experimental import pallas as pl
from jax.experimental.pallas import tpu as pltpu
from jax.experimental.pallas import tpu_sc as plsc

EMB = 50
BATCH = 16384
NC, NS, L = 2, 16, 16
NW = NC * NS
BPW = BATCH // NW
HALF = BPW // 2
HCHUNKS = HALF // L
IG = 128
NG = BPW // IG
NSEM = 8


def _sc_body(gidx_hbm, pidx_hbm, emb1_hbm, emb2_hbm, b1_hbm, b2_hbm,
             out_hbm, gidx_v, pidx_v, g_v, p_v, b1_v, b2_v,
             out_v, sem, *rsems):
    wid = lax.axis_index("s") * NC + lax.axis_index("c")
    base = wid * BPW

    for k in range(NG):
        pltpu.sync_copy(gidx_hbm.at[pl.ds(base + k * IG, IG)], gidx_v.at[k])
        pltpu.sync_copy(pidx_hbm.at[pl.ds(base + k * IG, IG)], pidx_v.at[k])

    bias_copies = []
    for k in range(NG):
        sl = pl.ds(k * IG, IG)
        bias_copies.append(pltpu.async_copy(
            b1_hbm.at[gidx_v.at[k]], b1_v.at[sl], sem))
        bias_copies.append(pltpu.async_copy(
            b2_hbm.at[pidx_v.at[k]], b2_v.at[sl], sem))
    for cp in bias_copies:
        cp.wait()

    lanes = lax.iota(jnp.int32, L)

    def half(h, carry):
        hbase = h * HALF

        def fire(grp, c):
            k = (hbase // IG) + grp // (IG // L)
            i = (grp % (IG // L)) * L
            gv16 = gidx_v[k, pl.ds(i, L)]
            pv16 = pidx_v[k, pl.ds(i, L)]
            r0 = grp * L
            for t in range(L):
                q = t % NSEM
                pltpu.async_copy(
                    emb1_hbm.at[gv16[t]], g_v.at[r0 + t], rsems[q])
                pltpu.async_copy(
                    emb2_hbm.at[pv16[t]], p_v.at[r0 + t], rsems[q])
            return c

        lax.fori_loop(0, HALF // L, fire, 0)

        per_sem = 2 * HALF // NSEM
        for q in range(NSEM):
            pltpu.make_async_copy(
                emb1_hbm.at[pl.ds(0, per_sem)],
                g_v.at[pl.ds(0, per_sem)], rsems[q]).wait()

        def chunk(c, carry2):
            r0 = c * L
            rows = r0 + lanes
            acc = (b1_v[pl.ds(hbase + r0, L)] + b2_v[pl.ds(hbase + r0, L)])
            for j in range(EMB):
                jv = jnp.full((L,), j, jnp.int32)
                gv = plsc.load_gather(g_v, [rows, jv])
                pv = plsc.load_gather(p_v, [rows, jv])
                acc = acc + gv * pv
            out_v[pl.ds(hbase + r0, L)] = acc
            return carry2

        lax.fori_loop(0, HCHUNKS, chunk, 0)
        return carry

    lax.fori_loop(0, 2, half, 0)

    pltpu.sync_copy(out_v, out_hbm.at[pl.ds(base, BPW)])


def _sc_call(gidx, pidx, emb_1, emb_2, b1, b2):
    mesh = plsc.VectorSubcoreMesh(core_axis_name="c", subcore_axis_name="s")
    k = functools.partial(
        pl.kernel,
        mesh=mesh,
        out_type=jax.ShapeDtypeStruct((BATCH,), jnp.float32),
        scratch_types=[
            pltpu.VMEM((NG, IG), jnp.int32),
            pltpu.VMEM((NG, IG), jnp.int32),
            pltpu.VMEM((HALF, EMB), jnp.float32),
            pltpu.VMEM((HALF, EMB), jnp.float32),
            pltpu.VMEM((BPW,), jnp.float32),
            pltpu.VMEM((BPW,), jnp.float32),
            pltpu.VMEM((BPW,), jnp.float32),
            pltpu.SemaphoreType.DMA,
        ] + [pltpu.SemaphoreType.DMA] * NSEM,
        compiler_params=pltpu.CompilerParams(needs_layout_passes=False),
    )(_sc_body)
    return k(gidx, pidx, emb_1, emb_2, b1, b2)


def kernel(x, emb_1, emb_2, emb_1_bias, emb_2_bias):
    gidx = x[:, 0].astype(jnp.int32)
    pidx = x[:, 1].astype(jnp.int32)
    b1 = emb_1_bias.reshape(-1)
    b2 = emb_2_bias.reshape(-1)
    return _sc_call(gidx, pidx, emb_1, emb_2, b1, b2)
